# baseline (speedup 1.0000x reference)
import functools

import jax
import jax.numpy as jnp
from jax import lax
from jax.experimental import pallas as pl
from jax.experimental.pallas import tpu as pltpu
from jax.experimental.pallas import tpu_sc as plsc

VOCAB = 1000000
DIM = 300
BATCH = 16384
MAX_NORM = 1.0

_NC = 2
_NS = 16
_NW = _NC * _NS
_BPW = BATCH // _NW
_CHUNK = 64
_NCHUNK = _BPW // _CHUNK
_L = 16
_NFULL = DIM // _L
_TAIL = DIM - _NFULL * _L
_TAIL_OFF = DIM - _L


def _sc_partials(center_idx, context_idx, W_center, W_context):
    mesh = plsc.VectorSubcoreMesh(core_axis_name="c", subcore_axis_name="s")

    @functools.partial(
        pl.kernel,
        out_type=(
            jax.ShapeDtypeStruct((128, BATCH * _L // 128), jnp.float32),
            jax.ShapeDtypeStruct((128, BATCH * _L // 128), jnp.float32),
            jax.ShapeDtypeStruct((128, BATCH * _L // 128), jnp.float32),
        ),
        mesh=mesh,
        compiler_params=pltpu.CompilerParams(use_tc_tiling_on_sc=False),
        scratch_types=[
            pltpu.VMEM((_NCHUNK, _CHUNK), jnp.int32),
            pltpu.VMEM((_NCHUNK, _CHUNK), jnp.int32),
            pltpu.VMEM((_CHUNK, DIM), jnp.float32),
            pltpu.VMEM((_CHUNK, DIM), jnp.float32),
            pltpu.VMEM((_CHUNK * _L,), jnp.float32),
            pltpu.VMEM((_CHUNK * _L,), jnp.float32),
            pltpu.VMEM((_CHUNK * _L,), jnp.float32),
            pltpu.SemaphoreType.DMA,
            pltpu.SemaphoreType.DMA,
        ],
    )
    def k(ci_hbm, xi_hbm, wc_hbm, wx_hbm, dot_hbm, c2_hbm, x2_hbm,
          ci_v, xi_v, rows_c, rows_x, dot_v, c2_v, x2_v, sem_c, sem_x):
        wid = lax.axis_index("s") * _NC + lax.axis_index("c")
        base = wid * _BPW

        pltpu.sync_copy(ci_hbm.at[wid], ci_v)
        pltpu.sync_copy(xi_hbm.at[wid], xi_v)

        tail_mask = lax.iota(jnp.int32, _L) >= (_L - _TAIL)

        def chunk_body(c, _):
            cp_c = pltpu.async_copy(wc_hbm.at[ci_v.at[c]], rows_c, sem_c)
            cp_x = pltpu.async_copy(wx_hbm.at[xi_v.at[c]], rows_x, sem_x)
            cp_c.wait()
            cp_x.wait()

            def row_body(r, _):
                dot = jnp.zeros((_L,), jnp.float32)
                cc = jnp.zeros((_L,), jnp.float32)
                xx = jnp.zeros((_L,), jnp.float32)
                for j in range(_NFULL):
                    cv = rows_c[r, pl.ds(j * _L, _L)]
                    xv = rows_x[r, pl.ds(j * _L, _L)]
                    dot = dot + cv * xv
                    cc = cc + cv * cv
                    xx = xx + xv * xv
                cv = rows_c[r, pl.ds(_TAIL_OFF, _L)]
                xv = rows_x[r, pl.ds(_TAIL_OFF, _L)]
                cv = jnp.where(tail_mask, cv, 0.0)
                xv = jnp.where(tail_mask, xv, 0.0)
                dot = dot + cv * xv
                cc = cc + cv * cv
                xx = xx + xv * xv
                dot_v[pl.ds(r * _L, _L)] = dot
                c2_v[pl.ds(r * _L, _L)] = cc
                x2_v[pl.ds(r * _L, _L)] = xx
                return 0

            lax.fori_loop(0, _CHUNK, row_body, 0)

            out_row = wid * (_BPW * _L // 2048) + c // 2
            out_col = (c % 2) * (_CHUNK * _L)
            dst = pl.ds(out_col, _CHUNK * _L)
            pltpu.sync_copy(dot_v, dot_hbm.at[out_row, dst])
            pltpu.sync_copy(c2_v, c2_hbm.at[out_row, dst])
            pltpu.sync_copy(x2_v, x2_hbm.at[out_row, dst])
            return 0

        lax.fori_loop(0, _NCHUNK, chunk_body, 0)

    ci = center_idx.reshape(_NW, _NCHUNK, _CHUNK)
    xi = context_idx.reshape(_NW, _NCHUNK, _CHUNK)
    return k(ci, xi, W_center, W_context)


def _tc_finish_body(dot_ref, c2_ref, x2_ref, out_ref):
    nl = BATCH * _L // 128
    li = lax.broadcasted_iota(jnp.int32, (nl, 128), 0)
    gi = lax.broadcasted_iota(jnp.int32, (nl, 128), 1)
    sel = (li // _L == gi).astype(jnp.float32)

    def fold(ref):
        return jnp.dot(ref[...], sel,
                       precision=lax.Precision.HIGHEST,
                       preferred_element_type=jnp.float32)

    dot = fold(dot_ref)
    c2 = fold(c2_ref)
    x2 = fold(x2_ref)
    scale_c = jnp.minimum(1.0, MAX_NORM / jnp.maximum(jnp.sqrt(c2), 1e-7))
    scale_x = jnp.minimum(1.0, MAX_NORM / jnp.maximum(jnp.sqrt(x2), 1e-7))
    s = dot * scale_c * scale_x
    loss = jax.nn.log_sigmoid(s)
    out_ref[...] = jnp.full((1, 1), -jnp.mean(loss), jnp.float32)


def kernel(center_input, context_input, W_center, W_context):
    ci = center_input.astype(jnp.int32)
    xi = context_input.astype(jnp.int32)
    dot_p, c2_p, x2_p = _sc_partials(ci, xi, W_center, W_context)
    res = pl.pallas_call(
        _tc_finish_body,
        out_shape=jax.ShapeDtypeStruct((1, 1), jnp.float32),
    )(dot_p, c2_p, x2_p)
    return res[0, 0]


# speedup vs baseline: 5.6925x; 5.6925x over previous
import functools

import jax
import jax.numpy as jnp
from jax import lax
from jax.experimental import pallas as pl
from jax.experimental.pallas import tpu as pltpu
from jax.experimental.pallas import tpu_sc as plsc

VOCAB = 1000000
DIM = 300
BATCH = 16384
MAX_NORM = 1.0

_NC = 2
_NS = 16
_NW = _NC * _NS
_BPW = BATCH // _NW
_L = 16
_DPAD = 384
_NG = (DIM + _L - 1) // _L
_RSTAGE = 16
_VB = 128
_VBMAX = VOCAB - _VB


def _sc_gather(idx, W_t):
    mesh = plsc.VectorSubcoreMesh(core_axis_name="c", subcore_axis_name="s")

    @functools.partial(
        pl.kernel,
        out_type=jax.ShapeDtypeStruct((BATCH, _DPAD), jnp.float32),
        mesh=mesh,
        compiler_params=pltpu.CompilerParams(
            use_tc_tiling_on_sc=True, needs_layout_passes=False),
        scratch_types=[
            pltpu.VMEM((_BPW + _L,), jnp.int32),
            pltpu.VMEM((DIM, _VB), jnp.float32),
            pltpu.VMEM((DIM, _VB), jnp.float32),
            pltpu.VMEM((_RSTAGE, _DPAD), jnp.float32),
            pltpu.SemaphoreType.DMA,
            pltpu.SemaphoreType.DMA,
        ],
    )
    def k(idx_hbm, wt_hbm, out_hbm, idx_v, blk0_v, blk1_v, rows_v, sem0, sem1):
        wid = lax.axis_index("s") * _NC + lax.axis_index("c")
        base = wid * _BPW

        pltpu.sync_copy(idx_hbm.at[wid], idx_v.at[pl.ds(0, _BPW)])

        lanes = lax.iota(jnp.int32, _L)
        sems = (sem0, sem1)
        blks = (blk0_v, blk1_v)

        def get_v(j):
            return idx_v[pl.ds(j, _L)][0]

        def start_fetch(j, buf):
            v = get_v(j)
            vb = pl.multiple_of(jnp.minimum((v // _VB) * _VB, _VBMAX), _VB)
            pltpu.async_copy(
                wt_hbm.at[:, pl.ds(vb, _VB)], blks[buf], sems[buf])

        def wait_fetch(buf):
            pltpu.make_async_copy(
                wt_hbm.at[:, pl.ds(0, _VB)], blks[buf], sems[buf]).wait()

        def extract(j, buf):
            v = get_v(j)
            vb = jnp.minimum((v // _VB) * _VB, _VBMAX)
            lane = v - vb
            lane_idx = jnp.full((_L,), lane, jnp.int32)
            r = j % _RSTAGE
            for g in range(_NG):
                row_idx = jnp.minimum(lanes + (g * _L), DIM - 1)
                col = plsc.load_gather(blks[buf], [row_idx, lane_idx])
                rows_v[r, pl.ds(g * _L, _L)] = col

        def flush(j):
            r0 = pl.multiple_of(base + j - (_RSTAGE - 1), _RSTAGE)
            pltpu.sync_copy(rows_v, out_hbm.at[pl.ds(r0, _RSTAGE)])

        start_fetch(0, 0)

        def body(step, _):
            for b in range(2):
                j = step * 2 + b
                nxt = jnp.minimum(j + 1, _BPW - 1)
                start_fetch(nxt, (b + 1) % 2)
                wait_fetch(b)
                extract(j, b)

                @pl.when(j % _RSTAGE == _RSTAGE - 1)
                def _():
                    flush(j)
            return 0

        lax.fori_loop(0, _BPW // 2, body, 0)
        wait_fetch(0)

    return k(idx.reshape(_NW, _BPW), W_t)


_FBLK = 2048


def _tc_finish_body(c_ref, x_ref, out_ref):
    d = lax.broadcasted_iota(jnp.int32, (1, _DPAD), 1)
    mask = (d < DIM).astype(jnp.float32)
    c = c_ref[...] * mask
    x = x_ref[...] * mask
    dot = jnp.sum(c * x, axis=1)
    c2 = jnp.sum(c * c, axis=1)
    x2 = jnp.sum(x * x, axis=1)
    scale_c = jnp.minimum(1.0, MAX_NORM / jnp.maximum(jnp.sqrt(c2), 1e-7))
    scale_x = jnp.minimum(1.0, MAX_NORM / jnp.maximum(jnp.sqrt(x2), 1e-7))
    s = dot * scale_c * scale_x
    loss = jax.nn.log_sigmoid(s)
    part = jnp.full((1, 1), -jnp.sum(loss) / BATCH, jnp.float32)

    @pl.when(pl.program_id(0) == 0)
    def _():
        out_ref[...] = jnp.zeros((1, 1), jnp.float32)

    out_ref[...] += part


def kernel(center_input, context_input, W_center, W_context):
    ci = center_input.astype(jnp.int32)
    xi = context_input.astype(jnp.int32)
    rows_c = _sc_gather(ci, W_center.T)
    rows_x = _sc_gather(xi, W_context.T)
    res = pl.pallas_call(
        _tc_finish_body,
        grid=(BATCH // _FBLK,),
        in_specs=[
            pl.BlockSpec((_FBLK, _DPAD), lambda i: (i, 0)),
            pl.BlockSpec((_FBLK, _DPAD), lambda i: (i, 0)),
        ],
        out_specs=pl.BlockSpec((1, 1), lambda i: (0, 0)),
        out_shape=jax.ShapeDtypeStruct((1, 1), jnp.float32),
    )(rows_c, rows_x)
    return res[0, 0]


# speedup vs baseline: 9.1972x; 1.6157x over previous
import functools

import jax
import jax.numpy as jnp
from jax import lax
from jax.experimental import pallas as pl
from jax.experimental.pallas import tpu as pltpu
from jax.experimental.pallas import tpu_sc as plsc

VOCAB = 1000000
DIM = 300
BATCH = 16384
MAX_NORM = 1.0

_NC = 2
_NS = 16
_NW = _NC * _NS
_BPW = BATCH // _NW
_L = 16
_DPAD = 384
_NG = (DIM + _L - 1) // _L
_RSTAGE = 16
_VB = 128
_VBMAX = VOCAB - _VB


def _sc_gather(idx_sorted, pos, W_t):
    mesh = plsc.VectorSubcoreMesh(core_axis_name="c", subcore_axis_name="s")

    @functools.partial(
        pl.kernel,
        out_type=jax.ShapeDtypeStruct((BATCH, _DPAD), jnp.float32),
        mesh=mesh,
        compiler_params=pltpu.CompilerParams(
            use_tc_tiling_on_sc=True, needs_layout_passes=False),
        scratch_types=[
            pltpu.VMEM((_BPW + _L,), jnp.int32),
            pltpu.VMEM((_BPW,), jnp.int32),
            pltpu.VMEM((DIM, _VB), jnp.float32),
            pltpu.VMEM((_RSTAGE, _DPAD), jnp.float32),
            pltpu.VMEM((_BPW // _RSTAGE, _L), jnp.int32),
            pltpu.SemaphoreType.DMA,
            pltpu.SemaphoreType.DMA,
        ],
    )
    def k(idx_hbm, pos_hbm, wt_hbm, out_hbm,
          idx_v, pos_v, blk_v, rows_v, spos_v, sem, osem):
        wid = lax.axis_index("s") * _NC + lax.axis_index("c")

        pltpu.sync_copy(idx_hbm.at[wid], idx_v.at[pl.ds(0, _BPW)])
        pltpu.sync_copy(pos_hbm.at[wid], pos_v)

        lanes = lax.iota(jnp.int32, _L)

        def body(j, vb_cur):
            v = idx_v[pl.ds(j, _L)][0]
            vb = pl.multiple_of(jnp.minimum((v // _VB) * _VB, _VBMAX), _VB)

            @pl.when(vb != vb_cur)
            def _():
                pltpu.async_copy(
                    wt_hbm.at[:, pl.ds(vb, _VB)], blk_v, sem).wait()

            lane_idx = jnp.full((_L,), v - vb, jnp.int32)
            r = j % _RSTAGE
            for g in range(_NG):
                row_idx = jnp.minimum(lanes + (g * _L), DIM - 1)
                rows_v[r, pl.ds(g * _L, _L)] = plsc.load_gather(
                    blk_v, [row_idx, lane_idx])

            @pl.when(r == _RSTAGE - 1)
            def _():
                f = j // _RSTAGE
                j0 = pl.multiple_of(j - (_RSTAGE - 1), _RSTAGE)
                spos_v[f, :] = pos_v[pl.ds(j0, _L)]
                pltpu.async_copy(rows_v, out_hbm.at[spos_v.at[f]], osem).wait()

            return vb

        lax.fori_loop(0, _BPW, body, jnp.int32(-1))

    return k(idx_sorted.reshape(_NW, _BPW), pos.reshape(_NW, _BPW), W_t)


_FBLK = 2048


def _tc_finish_body(c_ref, x_ref, out_ref):
    d = lax.broadcasted_iota(jnp.int32, (1, _DPAD), 1)
    mask = (d < DIM).astype(jnp.float32)
    c = c_ref[...] * mask
    x = x_ref[...] * mask
    dot = jnp.sum(c * x, axis=1)
    c2 = jnp.sum(c * c, axis=1)
    x2 = jnp.sum(x * x, axis=1)
    scale_c = jnp.minimum(1.0, MAX_NORM / jnp.maximum(jnp.sqrt(c2), 1e-7))
    scale_x = jnp.minimum(1.0, MAX_NORM / jnp.maximum(jnp.sqrt(x2), 1e-7))
    s = dot * scale_c * scale_x
    loss = jax.nn.log_sigmoid(s)
    part = jnp.full((1, 1), -jnp.sum(loss) / BATCH, jnp.float32)

    @pl.when(pl.program_id(0) == 0)
    def _():
        out_ref[...] = jnp.zeros((1, 1), jnp.float32)

    out_ref[...] += part


def kernel(center_input, context_input, W_center, W_context):
    ci = center_input.astype(jnp.int32)
    xi = context_input.astype(jnp.int32)
    iota = lax.iota(jnp.int32, BATCH)
    ci_s, ci_pos = lax.sort((ci, iota), num_keys=1)
    xi_s, xi_pos = lax.sort((xi, iota), num_keys=1)
    rows_c = _sc_gather(ci_s, ci_pos, W_center.T)
    rows_x = _sc_gather(xi_s, xi_pos, W_context.T)
    res = pl.pallas_call(
        _tc_finish_body,
        grid=(BATCH // _FBLK,),
        in_specs=[
            pl.BlockSpec((_FBLK, _DPAD), lambda i: (i, 0)),
            pl.BlockSpec((_FBLK, _DPAD), lambda i: (i, 0)),
        ],
        out_specs=pl.BlockSpec((1, 1), lambda i: (0, 0)),
        out_shape=jax.ShapeDtypeStruct((1, 1), jnp.float32),
    )(rows_c, rows_x)
    return res[0, 0]


# speedup vs baseline: 15.1972x; 1.6524x over previous
import functools

import jax
import jax.numpy as jnp
from jax import lax
from jax.experimental import pallas as pl
from jax.experimental.pallas import tpu as pltpu
from jax.experimental.pallas import tpu_sc as plsc

VOCAB = 1000000
DIM = 300
BATCH = 16384
MAX_NORM = 1.0

_NC = 2
_NS = 16
_NW = _NC * _NS
_BPW = BATCH // _NW
_L = 16
_DPAD = 384
_NG = (DIM + _L - 1) // _L
_RSTAGE = 16
_VB = 128
_VBMAX = VOCAB - _VB
_BSTRIDE = 304


def _sc_gather(idx_sorted, pos, W_t):
    mesh = plsc.VectorSubcoreMesh(core_axis_name="c", subcore_axis_name="s")

    @functools.partial(
        pl.kernel,
        out_type=jax.ShapeDtypeStruct((BATCH, _DPAD), jnp.float32),
        mesh=mesh,
        compiler_params=pltpu.CompilerParams(
            use_tc_tiling_on_sc=True, needs_layout_passes=False),
        scratch_types=[
            pltpu.VMEM((_BPW + _VB + _L,), jnp.int32),
            pltpu.VMEM((_BPW,), jnp.int32),
            pltpu.VMEM((_BPW + _L,), jnp.int32),
            pltpu.VMEM((2 * _BSTRIDE, _VB), jnp.float32),
            pltpu.VMEM((_RSTAGE, _DPAD), jnp.float32),
            pltpu.VMEM((_BPW // _RSTAGE, _L), jnp.int32),
            pltpu.SemaphoreType.DMA,
            pltpu.SemaphoreType.DMA,
        ],
    )
    def k(idx_hbm, pos_hbm, wt_hbm, out_hbm,
          idx_v, pos_v, dlist_v, blk_v, rows_v, spos_v, sem, osem):
        wid = lax.axis_index("s") * _NC + lax.axis_index("c")

        pltpu.sync_copy(idx_hbm.at[wid], idx_v.at[pl.ds(_VB, _BPW)])
        pltpu.sync_copy(pos_hbm.at[wid], pos_v)

        lanes = lax.iota(jnp.int32, _L)

        def blocks_of(vec):
            return jnp.minimum((vec // _VB) * _VB, _VBMAX)

        def scan_body(kc, nd):
            a = blocks_of(idx_v[pl.ds(_VB + kc * _L, _L)])
            b = blocks_of(idx_v[pl.ds(_VB - 1 + kc * _L, _L)])
            flags = (a != b) | ((lanes + kc) == 0)
            plsc.store_compressed(dlist_v.at[pl.ds(nd, _L)], a, mask=flags)
            return nd + plsc.all_reduce_population_count(flags)[0]

        nd = lax.fori_loop(0, _BPW // _L, scan_body, jnp.int32(0))

        def issue_fetch(b):
            vb = pl.multiple_of(dlist_v[pl.ds(b, _L)][0], _VB)
            off = pl.multiple_of((b % 2) * _BSTRIDE, 8)
            pltpu.async_copy(
                wt_hbm.at[:, pl.ds(vb, _VB)],
                blk_v.at[pl.ds(off, DIM)], sem)

        def wait_fetch():
            pltpu.make_async_copy(
                wt_hbm.at[:, pl.ds(0, _VB)],
                blk_v.at[pl.ds(0, DIM)], sem).wait()

        issue_fetch(jnp.int32(0))

        def body(j, carry):
            vb_cur, ordi = carry
            v = idx_v[pl.ds(_VB + j, _L)][0]
            vb = jnp.minimum((v // _VB) * _VB, _VBMAX)
            trans = vb != vb_cur
            ordn = jnp.where(trans, ordi + 1, ordi)

            @pl.when(trans)
            def _():
                @pl.when(ordn + 1 < nd)
                def _():
                    issue_fetch(ordn + 1)

                wait_fetch()

            off = (ordn % 2) * _BSTRIDE
            lane_idx = jnp.full((_L,), v - vb, jnp.int32)
            r = j % _RSTAGE
            for g in range(_NG):
                row_idx = jnp.minimum(lanes + (g * _L), DIM - 1) + off
                rows_v[r, pl.ds(g * _L, _L)] = plsc.load_gather(
                    blk_v, [row_idx, lane_idx])

            @pl.when(r == _RSTAGE - 1)
            def _():
                f = j // _RSTAGE
                j0 = pl.multiple_of(j - (_RSTAGE - 1), _RSTAGE)
                spos_v[f, :] = pos_v[pl.ds(j0, _L)]
                pltpu.async_copy(rows_v, out_hbm.at[spos_v.at[f]], osem).wait()

            return vb, ordn

        lax.fori_loop(0, _BPW, body, (jnp.int32(-1), jnp.int32(-1)))

    return k(idx_sorted.reshape(_NW, _BPW), pos.reshape(_NW, _BPW), W_t)


_FBLK = 2048


def _tc_finish_body(c_ref, x_ref, out_ref):
    d = lax.broadcasted_iota(jnp.int32, (1, _DPAD), 1)
    mask = (d < DIM).astype(jnp.float32)
    c = c_ref[...] * mask
    x = x_ref[...] * mask
    dot = jnp.sum(c * x, axis=1)
    c2 = jnp.sum(c * c, axis=1)
    x2 = jnp.sum(x * x, axis=1)
    scale_c = jnp.minimum(1.0, MAX_NORM / jnp.maximum(jnp.sqrt(c2), 1e-7))
    scale_x = jnp.minimum(1.0, MAX_NORM / jnp.maximum(jnp.sqrt(x2), 1e-7))
    s = dot * scale_c * scale_x
    loss = jax.nn.log_sigmoid(s)
    part = jnp.full((1, 1), -jnp.sum(loss) / BATCH, jnp.float32)

    @pl.when(pl.program_id(0) == 0)
    def _():
        out_ref[...] = jnp.zeros((1, 1), jnp.float32)

    out_ref[...] += part


def kernel(center_input, context_input, W_center, W_context):
    ci = center_input.astype(jnp.int32)
    xi = context_input.astype(jnp.int32)
    iota = lax.iota(jnp.int32, BATCH)
    ci_s, ci_pos = lax.sort((ci, iota), num_keys=1)
    xi_s, xi_pos = lax.sort((xi, iota), num_keys=1)
    rows_c = _sc_gather(ci_s, ci_pos, W_center.T)
    rows_x = _sc_gather(xi_s, xi_pos, W_context.T)
    res = pl.pallas_call(
        _tc_finish_body,
        grid=(BATCH // _FBLK,),
        in_specs=[
            pl.BlockSpec((_FBLK, _DPAD), lambda i: (i, 0)),
            pl.BlockSpec((_FBLK, _DPAD), lambda i: (i, 0)),
        ],
        out_specs=pl.BlockSpec((1, 1), lambda i: (0, 0)),
        out_shape=jax.ShapeDtypeStruct((1, 1), jnp.float32),
    )(rows_c, rows_x)
    return res[0, 0]


# speedup vs baseline: 15.4477x; 1.0165x over previous
import functools

import jax
import jax.numpy as jnp
from jax import lax
from jax.experimental import pallas as pl
from jax.experimental.pallas import tpu as pltpu
from jax.experimental.pallas import tpu_sc as plsc

VOCAB = 1000000
DIM = 300
BATCH = 16384
MAX_NORM = 1.0

_NC = 2
_NS = 16
_NW = _NC * _NS
_BPW = BATCH // _NW
_L = 16
_DPAD = 384
_NG = (DIM + _L - 1) // _L
_RSTAGE = 16
_VB = 128
_VBMAX = VOCAB - _VB
_BSTRIDE = 304


def _sc_gather2(ci_sorted, ci_pos, xi_sorted, xi_pos, Wc_t, Wx_t):
    mesh = plsc.VectorSubcoreMesh(core_axis_name="c", subcore_axis_name="s")

    @functools.partial(
        pl.kernel,
        out_type=(
            jax.ShapeDtypeStruct((BATCH, _DPAD), jnp.float32),
            jax.ShapeDtypeStruct((BATCH, _DPAD), jnp.float32),
        ),
        mesh=mesh,
        compiler_params=pltpu.CompilerParams(
            use_tc_tiling_on_sc=True, needs_layout_passes=False),
        scratch_types=[
            pltpu.VMEM((_BPW + _VB + _L,), jnp.int32),
            pltpu.VMEM((_BPW,), jnp.int32),
            pltpu.VMEM((_BPW + _L,), jnp.int32),
            pltpu.VMEM((2 * _BSTRIDE, _VB), jnp.float32),
            pltpu.VMEM((_RSTAGE, _DPAD), jnp.float32),
            pltpu.VMEM((_BPW // _RSTAGE, _L), jnp.int32),
            pltpu.SemaphoreType.DMA,
            pltpu.SemaphoreType.DMA,
        ],
    )
    def k(ci_hbm, cp_hbm, xi_hbm, xp_hbm, wc_hbm, wx_hbm, outc_hbm, outx_hbm,
          idx_v, pos_v, dlist_v, blk_v, rows_v, spos_v, sem, osem):
        wid = lax.axis_index("s") * _NC + lax.axis_index("c")

        lanes = lax.iota(jnp.int32, _L)

        def blocks_of(vec):
            return jnp.minimum((vec // _VB) * _VB, _VBMAX)

        def gather_one(idx_hbm, pos_hbm, wt_hbm, out_hbm):
            pltpu.sync_copy(idx_hbm.at[wid], idx_v.at[pl.ds(_VB, _BPW)])
            pltpu.sync_copy(pos_hbm.at[wid], pos_v)

            def scan_body(kc, nd):
                a = blocks_of(idx_v[pl.ds(_VB + kc * _L, _L)])
                b = blocks_of(idx_v[pl.ds(_VB - 1 + kc * _L, _L)])
                flags = (a != b) | ((lanes + kc) == 0)
                plsc.store_compressed(dlist_v.at[pl.ds(nd, _L)], a, mask=flags)
                return nd + plsc.all_reduce_population_count(flags)[0]

            nd = lax.fori_loop(0, _BPW // _L, scan_body, jnp.int32(0))

            def issue_fetch(b):
                vb = pl.multiple_of(dlist_v[pl.ds(b, _L)][0], _VB)
                off = pl.multiple_of((b % 2) * _BSTRIDE, 8)
                pltpu.async_copy(
                    wt_hbm.at[:, pl.ds(vb, _VB)],
                    blk_v.at[pl.ds(off, DIM)], sem)

            def wait_fetch():
                pltpu.make_async_copy(
                    wt_hbm.at[:, pl.ds(0, _VB)],
                    blk_v.at[pl.ds(0, DIM)], sem).wait()

            issue_fetch(jnp.int32(0))

            def body(j, carry):
                vb_cur, ordi = carry
                v = idx_v[pl.ds(_VB + j, _L)][0]
                vb = jnp.minimum((v // _VB) * _VB, _VBMAX)
                trans = vb != vb_cur
                ordn = jnp.where(trans, ordi + 1, ordi)

                @pl.when(trans)
                def _():
                    @pl.when(ordn + 1 < nd)
                    def _():
                        issue_fetch(ordn + 1)

                    wait_fetch()

                off = (ordn % 2) * _BSTRIDE
                lane_idx = jnp.full((_L,), v - vb, jnp.int32)
                r = j % _RSTAGE
                for g in range(_NG):
                    row_idx = jnp.minimum(lanes + (g * _L), DIM - 1) + off
                    rows_v[r, pl.ds(g * _L, _L)] = plsc.load_gather(
                        blk_v, [row_idx, lane_idx])

                @pl.when(r == _RSTAGE - 1)
                def _():
                    f = j // _RSTAGE
                    j0 = pl.multiple_of(j - (_RSTAGE - 1), _RSTAGE)
                    spos_v[f, :] = pos_v[pl.ds(j0, _L)]
                    pltpu.async_copy(
                        rows_v, out_hbm.at[spos_v.at[f]], osem).wait()

                return vb, ordn

            lax.fori_loop(0, _BPW, body, (jnp.int32(-1), jnp.int32(-1)))

        gather_one(ci_hbm, cp_hbm, wc_hbm, outc_hbm)
        gather_one(xi_hbm, xp_hbm, wx_hbm, outx_hbm)

    return k(ci_sorted.reshape(_NW, _BPW), ci_pos.reshape(_NW, _BPW),
             xi_sorted.reshape(_NW, _BPW), xi_pos.reshape(_NW, _BPW),
             Wc_t, Wx_t)


_FBLK = 2048


def _tc_finish_body(c_ref, x_ref, out_ref):
    d = lax.broadcasted_iota(jnp.int32, (1, _DPAD), 1)
    mask = (d < DIM).astype(jnp.float32)
    c = c_ref[...] * mask
    x = x_ref[...] * mask
    dot = jnp.sum(c * x, axis=1)
    c2 = jnp.sum(c * c, axis=1)
    x2 = jnp.sum(x * x, axis=1)
    scale_c = jnp.minimum(1.0, MAX_NORM / jnp.maximum(jnp.sqrt(c2), 1e-7))
    scale_x = jnp.minimum(1.0, MAX_NORM / jnp.maximum(jnp.sqrt(x2), 1e-7))
    s = dot * scale_c * scale_x
    loss = jax.nn.log_sigmoid(s)
    part = jnp.full((1, 1), -jnp.sum(loss) / BATCH, jnp.float32)

    @pl.when(pl.program_id(0) == 0)
    def _():
        out_ref[...] = jnp.zeros((1, 1), jnp.float32)

    out_ref[...] += part


def kernel(center_input, context_input, W_center, W_context):
    ci = center_input.astype(jnp.int32)
    xi = context_input.astype(jnp.int32)
    iota = lax.iota(jnp.int32, BATCH)
    ci_s, ci_pos = lax.sort((ci, iota), num_keys=1)
    xi_s, xi_pos = lax.sort((xi, iota), num_keys=1)
    rows_c, rows_x = _sc_gather2(
        ci_s, ci_pos, xi_s, xi_pos, W_center.T, W_context.T)
    res = pl.pallas_call(
        _tc_finish_body,
        grid=(BATCH // _FBLK,),
        in_specs=[
            pl.BlockSpec((_FBLK, _DPAD), lambda i: (i, 0)),
            pl.BlockSpec((_FBLK, _DPAD), lambda i: (i, 0)),
        ],
        out_specs=pl.BlockSpec((1, 1), lambda i: (0, 0)),
        out_shape=jax.ShapeDtypeStruct((1, 1), jnp.float32),
    )(rows_c, rows_x)
    return res[0, 0]


# speedup vs baseline: 15.9410x; 1.0319x over previous
import functools

import jax
import jax.numpy as jnp
from jax import lax
from jax.experimental import pallas as pl
from jax.experimental.pallas import tpu as pltpu
from jax.experimental.pallas import tpu_sc as plsc

VOCAB = 1000000
DIM = 300
BATCH = 16384
MAX_NORM = 1.0

_NC = 2
_NS = 16
_NW = _NC * _NS
_BPW = BATCH // _NW
_L = 16
_DPAD = 384
_NG = (DIM + _L - 1) // _L
_RSTAGE = 16
_VB = 128
_VBMAX = VOCAB - _VB
_BSTRIDE = 304


def _sc_gather2(ci_sorted, ci_pos, xi_sorted, xi_pos, Wc_t, Wx_t):
    mesh = plsc.VectorSubcoreMesh(core_axis_name="c", subcore_axis_name="s")

    @functools.partial(
        pl.kernel,
        out_type=(
            jax.ShapeDtypeStruct((BATCH, _DPAD), jnp.float32),
            jax.ShapeDtypeStruct((BATCH, _DPAD), jnp.float32),
        ),
        mesh=mesh,
        compiler_params=pltpu.CompilerParams(
            use_tc_tiling_on_sc=True, needs_layout_passes=False),
        scratch_types=[
            pltpu.VMEM((_BPW + _VB + _L,), jnp.int32),
            pltpu.VMEM((_BPW,), jnp.int32),
            pltpu.VMEM((_BPW + _L,), jnp.int32),
            pltpu.VMEM((3 * _BSTRIDE, _VB), jnp.float32),
            pltpu.VMEM((_RSTAGE, _DPAD), jnp.float32),
            pltpu.VMEM((_BPW // _RSTAGE, _L), jnp.int32),
            pltpu.SemaphoreType.DMA,
            pltpu.SemaphoreType.DMA,
        ],
    )
    def k(ci_hbm, cp_hbm, xi_hbm, xp_hbm, wc_hbm, wx_hbm, outc_hbm, outx_hbm,
          idx_v, pos_v, dlist_v, blk_v, rows_v, spos_v, sem, osem):
        wid = lax.axis_index("s") * _NC + lax.axis_index("c")

        lanes = lax.iota(jnp.int32, _L)

        def blocks_of(vec):
            return jnp.minimum((vec // _VB) * _VB, _VBMAX)

        def gather_one(idx_hbm, pos_hbm, wt_hbm, out_hbm):
            pltpu.sync_copy(idx_hbm.at[wid], idx_v.at[pl.ds(_VB, _BPW)])
            pltpu.sync_copy(pos_hbm.at[wid], pos_v)

            def scan_body(kc, nd):
                a = blocks_of(idx_v[pl.ds(_VB + kc * _L, _L)])
                b = blocks_of(idx_v[pl.ds(_VB - 1 + kc * _L, _L)])
                flags = (a != b) | ((lanes + kc) == 0)
                plsc.store_compressed(dlist_v.at[pl.ds(nd, _L)], a, mask=flags)
                return nd + plsc.all_reduce_population_count(flags)[0]

            nd = lax.fori_loop(0, _BPW // _L, scan_body, jnp.int32(0))

            def issue_fetch(b):
                vb = pl.multiple_of(dlist_v[pl.ds(b, _L)][0], _VB)
                off = pl.multiple_of((b % 3) * _BSTRIDE, 8)
                pltpu.async_copy(
                    wt_hbm.at[:, pl.ds(vb, _VB)],
                    blk_v.at[pl.ds(off, DIM)], sem)

            def wait_fetch():
                pltpu.make_async_copy(
                    wt_hbm.at[:, pl.ds(0, _VB)],
                    blk_v.at[pl.ds(0, DIM)], sem).wait()

            issue_fetch(jnp.int32(0))

            @pl.when(nd > 1)
            def _():
                issue_fetch(jnp.int32(1))

            def body(j, carry):
                vb_cur, ordi = carry
                v = idx_v[pl.ds(_VB + j, _L)][0]
                vb = jnp.minimum((v // _VB) * _VB, _VBMAX)
                trans = vb != vb_cur
                ordn = jnp.where(trans, ordi + 1, ordi)

                @pl.when(trans)
                def _():
                    @pl.when(ordn + 2 < nd)
                    def _():
                        issue_fetch(ordn + 2)

                    wait_fetch()

                off = (ordn % 3) * _BSTRIDE
                lane_idx = jnp.full((_L,), v - vb, jnp.int32)
                r = j % _RSTAGE
                for g in range(_NG):
                    row_idx = jnp.minimum(lanes + (g * _L), DIM - 1) + off
                    rows_v[r, pl.ds(g * _L, _L)] = plsc.load_gather(
                        blk_v, [row_idx, lane_idx])

                @pl.when(r == _RSTAGE - 1)
                def _():
                    f = j // _RSTAGE
                    j0 = pl.multiple_of(j - (_RSTAGE - 1), _RSTAGE)
                    spos_v[f, :] = pos_v[pl.ds(j0, _L)]
                    pltpu.async_copy(
                        rows_v, out_hbm.at[spos_v.at[f]], osem).wait()

                return vb, ordn

            lax.fori_loop(0, _BPW, body, (jnp.int32(-1), jnp.int32(-1)))

        gather_one(ci_hbm, cp_hbm, wc_hbm, outc_hbm)
        gather_one(xi_hbm, xp_hbm, wx_hbm, outx_hbm)

    return k(ci_sorted.reshape(_NW, _BPW), ci_pos.reshape(_NW, _BPW),
             xi_sorted.reshape(_NW, _BPW), xi_pos.reshape(_NW, _BPW),
             Wc_t, Wx_t)


_FBLK = 2048


def _tc_finish_body(c_ref, x_ref, out_ref):
    d = lax.broadcasted_iota(jnp.int32, (1, _DPAD), 1)
    mask = (d < DIM).astype(jnp.float32)
    c = c_ref[...] * mask
    x = x_ref[...] * mask
    dot = jnp.sum(c * x, axis=1)
    c2 = jnp.sum(c * c, axis=1)
    x2 = jnp.sum(x * x, axis=1)
    scale_c = jnp.minimum(1.0, MAX_NORM / jnp.maximum(jnp.sqrt(c2), 1e-7))
    scale_x = jnp.minimum(1.0, MAX_NORM / jnp.maximum(jnp.sqrt(x2), 1e-7))
    s = dot * scale_c * scale_x
    loss = jax.nn.log_sigmoid(s)
    part = jnp.full((1, 1), -jnp.sum(loss) / BATCH, jnp.float32)

    @pl.when(pl.program_id(0) == 0)
    def _():
        out_ref[...] = jnp.zeros((1, 1), jnp.float32)

    out_ref[...] += part


def kernel(center_input, context_input, W_center, W_context):
    ci = center_input.astype(jnp.int32)
    xi = context_input.astype(jnp.int32)
    iota = lax.iota(jnp.int32, BATCH)
    ci_s, ci_pos = lax.sort((ci, iota), num_keys=1)
    xi_s, xi_pos = lax.sort((xi, iota), num_keys=1)
    rows_c, rows_x = _sc_gather2(
        ci_s, ci_pos, xi_s, xi_pos, W_center.T, W_context.T)
    res = pl.pallas_call(
        _tc_finish_body,
        grid=(BATCH // _FBLK,),
        in_specs=[
            pl.BlockSpec((_FBLK, _DPAD), lambda i: (i, 0)),
            pl.BlockSpec((_FBLK, _DPAD), lambda i: (i, 0)),
        ],
        out_specs=pl.BlockSpec((1, 1), lambda i: (0, 0)),
        out_shape=jax.ShapeDtypeStruct((1, 1), jnp.float32),
    )(rows_c, rows_x)
    return res[0, 0]
